# baseline (speedup 1.0000x reference)
from typing import Any

import jax
import jax.numpy as jnp
import numpy as np
from jax.experimental import pallas as pl
from jax.experimental.pallas import tpu as pltpu

_N = 4096
_BATCH = 2
_EMBED = 32
_NUM_LAYERS = 4
_BM = 256
_FE = _BATCH * _EMBED
_PREC = jax.lax.Precision.DEFAULT
_BN_DEN = np.sqrt(1.0 + 1e-5)
_F32 = jnp.float32


def _scaling_of(r):
    s = jnp.sqrt(jnp.sum(r * r, axis=0, keepdims=True)) / np.sqrt(_N)
    return jnp.where(s < 1e-12, jnp.float32(1.0), s)


def _dg(a, b, dims, prec=_PREC):
    return jax.lax.dot_general(a, b, (dims, ((), ())), precision=prec,
                               preferred_element_type=_F32)


def _encode_body(r_ref, w0_ref, b0_ref, w1_ref, b1_ref, w2_ref, b2_ref,
                 w3_ref, b3_ref, out_ref):
    r = r_ref[...]
    rs = r / _scaling_of(r)
    for b in range(_BATCH):
        x = rs[:, b:b + 1]
        h = jax.nn.relu(x * w0_ref[...] + b0_ref[...])
        h = jax.nn.relu(jnp.dot(h, w1_ref[...], precision=_PREC,
                                preferred_element_type=_F32) + b1_ref[...])
        h = jax.nn.relu(jnp.dot(h, w2_ref[...], precision=_PREC,
                                preferred_element_type=_F32) + b2_ref[...])
        h = jax.nn.relu(jnp.dot(h, w3_ref[...], precision=_PREC,
                                preferred_element_type=_F32) + b3_ref[...])
        out_ref[:, b * _EMBED:(b + 1) * _EMBED] = h


def _main_body(aa_ref, r0_ref, wg_ref, ws_ref, b_ref, g_ref, b2_ref,
               r_small_ref, dw0_ref, db0_ref, dw1_ref, db1_ref, dw2_ref,
               db2_ref, dw3_ref, db3_ref, zt_ref,
               aabf_ref, rta_ref, rtb_ref, rta_bf_ref, rtb_bf_ref,
               stage_ref, sem_ref):
    i = pl.program_id(0)
    j = pl.program_id(1)
    cols = pl.ds(j * _BM, _BM)

    def _layer(gt, st, dst32, dstbf):
        y = ((_dg(wg_ref[0], gt, ((0,), (0,))) + b_ref[0]) + st)
        y = (y * g_ref[0]) / _BN_DEN + b2_ref[0]
        rnew = jax.nn.relu(y)
        dst32[:, cols] = rnew
        dstbf[:, cols] = rnew.astype(jnp.bfloat16)
        return rnew

    def _decode(rnew):
        r = r_small_ref[...]
        scaling = _scaling_of(r)
        for b in range(_BATCH):
            h = rnew[b * _EMBED:(b + 1) * _EMBED, :]
            h = jax.nn.relu(_dg(dw0_ref[...], h, ((0,), (0,)))
                            + db0_ref[...])
            h = jax.nn.relu(_dg(dw1_ref[...], h, ((0,), (0,)))
                            + db1_ref[...])
            h = jax.nn.relu(_dg(dw2_ref[...], h, ((0,), (0,)))
                            + db2_ref[...])
            z = _dg(dw3_ref[...], h, ((0,), (0,))) + db3_ref[...]
            zt_ref[b:b + 1, :] = z * scaling[0, b]

    @pl.when(i == 0)
    def _l0():
        cp = pltpu.make_async_copy(aa_ref.at[pl.ds(j * _BM, _BM), :],
                                   stage_ref, sem_ref)
        cp.start()
        cp.wait()
        aablk = stage_ref[...]
        aabf_ref[pl.ds(j * _BM, _BM), :] = aablk.astype(jnp.bfloat16)
        gt = _dg(r0_ref[...], aablk, ((0,), (1,)))
        st = _dg(ws_ref[0], r0_ref[cols, :], ((0,), (1,)))
        _layer(gt, st, rta_ref, rta_bf_ref)

    @pl.when(i == 1)
    def _l1():
        gt = _dg(rta_bf_ref[...], aabf_ref[pl.ds(j * _BM, _BM), :],
                 ((1,), (1,)))
        st = _dg(ws_ref[0], rta_ref[:, cols], ((0,), (0,)))
        _layer(gt, st, rtb_ref, rtb_bf_ref)

    @pl.when(i == 2)
    def _l2():
        gt = _dg(rtb_bf_ref[...], aabf_ref[pl.ds(j * _BM, _BM), :],
                 ((1,), (1,)))
        st = _dg(ws_ref[0], rtb_ref[:, cols], ((0,), (0,)))
        _layer(gt, st, rta_ref, rta_bf_ref)

    @pl.when(i == 3)
    def _l3():
        gt = _dg(rta_bf_ref[...], aabf_ref[pl.ds(j * _BM, _BM), :],
                 ((1,), (1,)))
        st = _dg(ws_ref[0], rta_ref[:, cols], ((0,), (0,)))
        rnew = _layer(gt, st, rtb_ref, rtb_bf_ref)
        _decode(rnew)


def _full(shape):
    return pl.BlockSpec(shape, lambda *a: (0,) * len(shape))


@jax.jit
def kernel(r, AA, params: dict[str, Any]):
    def bdiag(w):
        z = jnp.zeros_like(w)
        return jnp.block([[w, z], [z, w]])

    lwg, lws, lb, lg, lb2 = [], [], [], [], []
    for i in range(_NUM_LAYERS):
        lwg.append(bdiag(params["gc_W"][i]))
        lws.append(bdiag(params["sk_W"][i]))
        bb = (jnp.concatenate([params["gc_b"][i]] * _BATCH)
              + jnp.concatenate([params["sk_b"][i]] * _BATCH))
        lb.append(bb[:, None])
        lg.append(jnp.concatenate([params["bn_g"][i]] * _BATCH)[:, None])
        lb2.append(jnp.concatenate([params["bn_b"][i]] * _BATCH)[:, None])

    mi_W, mi_b = params["mi_W"], [b[None, :] for b in params["mi_b"]]
    mf_W = params["mf_W"]
    mf_b = [b[:, None] for b in params["mf_b"]]

    enc_args, enc_specs = [r], [_full((_N, _BATCH))]
    for W, b in zip(mi_W, mi_b):
        enc_args += [W, b]
        enc_specs += [_full(W.shape), _full(b.shape)]
    R0 = pl.pallas_call(
        _encode_body, grid=(1,), in_specs=enc_specs,
        out_specs=_full((_N, _FE)),
        out_shape=jax.ShapeDtypeStruct((_N, _FE), _F32),
    )(*enc_args)

    nb = _N // _BM
    main_args = [AA, R0, jnp.stack(lwg), jnp.stack(lws), jnp.stack(lb),
                 jnp.stack(lg), jnp.stack(lb2), r]
    main_specs = [
        pl.BlockSpec(memory_space=pltpu.MemorySpace.HBM),
        _full((_N, _FE)),
        pl.BlockSpec((1, _FE, _FE), lambda i, j: (i, 0, 0)),
        pl.BlockSpec((1, _FE, _FE), lambda i, j: (i, 0, 0)),
        pl.BlockSpec((1, _FE, 1), lambda i, j: (i, 0, 0)),
        pl.BlockSpec((1, _FE, 1), lambda i, j: (i, 0, 0)),
        pl.BlockSpec((1, _FE, 1), lambda i, j: (i, 0, 0)),
        _full((_N, _BATCH)),
    ]
    for W, b in zip(mf_W, mf_b):
        main_args += [W, b]
        main_specs += [_full(W.shape), _full(b.shape)]
    zt = pl.pallas_call(
        _main_body, grid=(_NUM_LAYERS, nb), in_specs=main_specs,
        out_specs=pl.BlockSpec((_BATCH, _BM), lambda i, j: (0, j)),
        out_shape=jax.ShapeDtypeStruct((_BATCH, _N), _F32),
        scratch_shapes=[pltpu.VMEM((_N, _N), jnp.bfloat16),
                        pltpu.VMEM((_FE, _N), _F32),
                        pltpu.VMEM((_FE, _N), _F32),
                        pltpu.VMEM((_FE, _N), jnp.bfloat16),
                        pltpu.VMEM((_FE, _N), jnp.bfloat16),
                        pltpu.VMEM((_BM, _N), _F32),
                        pltpu.SemaphoreType.DMA],
    )(*main_args)
    return zt.T


# speedup vs baseline: 1.2014x; 1.2014x over previous
from typing import Any

import jax
import jax.numpy as jnp
import numpy as np
from jax.experimental import pallas as pl
from jax.experimental.pallas import tpu as pltpu

_N = 4096
_BATCH = 2
_EMBED = 32
_NUM_LAYERS = 4
_BM = 256
_FE = _BATCH * _EMBED
_PREC = jax.lax.Precision.DEFAULT
_BN_DEN = np.sqrt(1.0 + 1e-5)
_F32 = jnp.float32


def _scaling_of(r):
    s = jnp.sqrt(jnp.sum(r * r, axis=0, keepdims=True)) / np.sqrt(_N)
    return jnp.where(s < 1e-12, jnp.float32(1.0), s)


def _dg(a, b, dims, prec=_PREC):
    return jax.lax.dot_general(a, b, (dims, ((), ())), precision=prec,
                               preferred_element_type=_F32)


def _encode_body(r_ref, w0_ref, b0_ref, w1_ref, b1_ref, w2_ref, b2_ref,
                 w3_ref, b3_ref, out_ref):
    r = r_ref[...]
    rs = r / _scaling_of(r)
    for b in range(_BATCH):
        x = rs[:, b:b + 1]
        h = jax.nn.relu(x * w0_ref[...] + b0_ref[...])
        h = jax.nn.relu(jnp.dot(h, w1_ref[...], precision=_PREC,
                                preferred_element_type=_F32) + b1_ref[...])
        h = jax.nn.relu(jnp.dot(h, w2_ref[...], precision=_PREC,
                                preferred_element_type=_F32) + b2_ref[...])
        h = jax.nn.relu(jnp.dot(h, w3_ref[...], precision=_PREC,
                                preferred_element_type=_F32) + b3_ref[...])
        out_ref[:, b * _EMBED:(b + 1) * _EMBED] = h


def _main_body(aa_ref, r0_ref, wg_ref, ws_ref, b_ref, g_ref, b2_ref,
               r_small_ref, dw0_ref, db0_ref, dw1_ref, db1_ref, dw2_ref,
               db2_ref, dw3_ref, db3_ref, zt_ref,
               aabf_ref, rta_ref, rtb_ref, rta_bf_ref, rtb_bf_ref):
    i = pl.program_id(0)
    j = pl.program_id(1)
    cols = pl.ds(j * _BM, _BM)

    def _layer(gt, st, dst32, dstbf):
        y = ((_dg(wg_ref[0], gt, ((0,), (0,))) + b_ref[0]) + st)
        y = (y * g_ref[0]) / _BN_DEN + b2_ref[0]
        rnew = jax.nn.relu(y)
        dst32[:, cols] = rnew
        dstbf[:, cols] = rnew.astype(jnp.bfloat16)
        return rnew

    def _decode(rnew):
        r = r_small_ref[...]
        scaling = _scaling_of(r)
        for b in range(_BATCH):
            h = rnew[b * _EMBED:(b + 1) * _EMBED, :]
            h = jax.nn.relu(_dg(dw0_ref[...], h, ((0,), (0,)))
                            + db0_ref[...])
            h = jax.nn.relu(_dg(dw1_ref[...], h, ((0,), (0,)))
                            + db1_ref[...])
            h = jax.nn.relu(_dg(dw2_ref[...], h, ((0,), (0,)))
                            + db2_ref[...])
            z = _dg(dw3_ref[...], h, ((0,), (0,))) + db3_ref[...]
            zt_ref[b:b + 1, :] = z * scaling[0, b]

    @pl.when(i == 0)
    def _l0():
        aablk = aa_ref[...]
        aabf_ref[pl.ds(j * _BM, _BM), :] = aablk.astype(jnp.bfloat16)
        gt = _dg(r0_ref[...], aablk, ((0,), (1,)))
        st = _dg(ws_ref[0], r0_ref[cols, :], ((0,), (1,)))
        _layer(gt, st, rta_ref, rta_bf_ref)

    @pl.when(i == 1)
    def _l1():
        gt = _dg(rta_bf_ref[...], aabf_ref[pl.ds(j * _BM, _BM), :],
                 ((1,), (1,)))
        st = _dg(ws_ref[0], rta_ref[:, cols], ((0,), (0,)))
        _layer(gt, st, rtb_ref, rtb_bf_ref)

    @pl.when(i == 2)
    def _l2():
        gt = _dg(rtb_bf_ref[...], aabf_ref[pl.ds(j * _BM, _BM), :],
                 ((1,), (1,)))
        st = _dg(ws_ref[0], rtb_ref[:, cols], ((0,), (0,)))
        _layer(gt, st, rta_ref, rta_bf_ref)

    @pl.when(i == 3)
    def _l3():
        gt = _dg(rta_bf_ref[...], aabf_ref[pl.ds(j * _BM, _BM), :],
                 ((1,), (1,)))
        st = _dg(ws_ref[0], rta_ref[:, cols], ((0,), (0,)))
        rnew = _layer(gt, st, rtb_ref, rtb_bf_ref)
        _decode(rnew)


def _full(shape):
    return pl.BlockSpec(shape, lambda *a: (0,) * len(shape))


@jax.jit
def kernel(r, AA, params: dict[str, Any]):
    def bdiag(w):
        z = jnp.zeros_like(w)
        return jnp.block([[w, z], [z, w]])

    lwg, lws, lb, lg, lb2 = [], [], [], [], []
    for i in range(_NUM_LAYERS):
        lwg.append(bdiag(params["gc_W"][i]))
        lws.append(bdiag(params["sk_W"][i]))
        bb = (jnp.concatenate([params["gc_b"][i]] * _BATCH)
              + jnp.concatenate([params["sk_b"][i]] * _BATCH))
        lb.append(bb[:, None])
        lg.append(jnp.concatenate([params["bn_g"][i]] * _BATCH)[:, None])
        lb2.append(jnp.concatenate([params["bn_b"][i]] * _BATCH)[:, None])

    mi_W, mi_b = params["mi_W"], [b[None, :] for b in params["mi_b"]]
    mf_W = params["mf_W"]
    mf_b = [b[:, None] for b in params["mf_b"]]

    enc_args, enc_specs = [r], [_full((_N, _BATCH))]
    for W, b in zip(mi_W, mi_b):
        enc_args += [W, b]
        enc_specs += [_full(W.shape), _full(b.shape)]
    R0 = pl.pallas_call(
        _encode_body, grid=(1,), in_specs=enc_specs,
        out_specs=_full((_N, _FE)),
        out_shape=jax.ShapeDtypeStruct((_N, _FE), _F32),
    )(*enc_args)

    nb = _N // _BM
    main_args = [AA, R0, jnp.stack(lwg), jnp.stack(lws), jnp.stack(lb),
                 jnp.stack(lg), jnp.stack(lb2), r]
    main_specs = [
        pl.BlockSpec((_BM, _N),
                     lambda i, j: (jnp.where(i == 0, j, nb - 1), 0)),
        _full((_N, _FE)),
        pl.BlockSpec((1, _FE, _FE), lambda i, j: (i, 0, 0)),
        pl.BlockSpec((1, _FE, _FE), lambda i, j: (i, 0, 0)),
        pl.BlockSpec((1, _FE, 1), lambda i, j: (i, 0, 0)),
        pl.BlockSpec((1, _FE, 1), lambda i, j: (i, 0, 0)),
        pl.BlockSpec((1, _FE, 1), lambda i, j: (i, 0, 0)),
        _full((_N, _BATCH)),
    ]
    for W, b in zip(mf_W, mf_b):
        main_args += [W, b]
        main_specs += [_full(W.shape), _full(b.shape)]
    zt = pl.pallas_call(
        _main_body, grid=(_NUM_LAYERS, nb), in_specs=main_specs,
        out_specs=pl.BlockSpec((_BATCH, _BM), lambda i, j: (0, j)),
        out_shape=jax.ShapeDtypeStruct((_BATCH, _N), _F32),
        scratch_shapes=[pltpu.VMEM((_N, _N), jnp.bfloat16),
                        pltpu.VMEM((_FE, _N), _F32),
                        pltpu.VMEM((_FE, _N), _F32),
                        pltpu.VMEM((_FE, _N), jnp.bfloat16),
                        pltpu.VMEM((_FE, _N), jnp.bfloat16)],
    )(*main_args)
    return zt.T
